# baseline (speedup 1.0000x reference)
import functools

import jax
import jax.numpy as jnp
from jax import lax
from jax.experimental import pallas as pl
from jax.experimental.pallas import tpu as pltpu
from jax.experimental.pallas import tpu_sc as plsc

VOCAB_DIM = 1000000
EMB = 64
SEQ = 200

_info = plsc.get_sparse_core_info()
_NC, _NS = _info.num_cores, _info.num_subcores
_ROWS_PER_W = 8
_N_ACTIVE = SEQ // _ROWS_PER_W


@functools.partial(
    pl.kernel,
    mesh=plsc.VectorSubcoreMesh(core_axis_name="c", subcore_axis_name="s"),
    out_type=jax.ShapeDtypeStruct((SEQ, EMB), jnp.float32),
    scratch_types=[
        pltpu.VMEM((16,), jnp.int32),
        pltpu.VMEM((_ROWS_PER_W, EMB), jnp.float32),
        pltpu.SemaphoreType.DMA,
    ],
    compiler_params=pltpu.CompilerParams(use_tc_tiling_on_sc=False),
)
def _sc_gather(table_hbm, idx_hbm, out_hbm, idx_v, rows_v, sem):
    wid = lax.axis_index("s") * _NC + lax.axis_index("c")

    @pl.when(wid < _N_ACTIVE)
    def _():
        base = wid * _ROWS_PER_W
        pltpu.sync_copy(
            idx_hbm.at[pl.ds(base, _ROWS_PER_W)], idx_v.at[pl.ds(0, _ROWS_PER_W)]
        )
        idx_vec = idx_v[...]
        copies = []
        for j in range(_ROWS_PER_W):
            row = idx_vec[j]
            copies.append(
                pltpu.async_copy(
                    table_hbm.at[pl.ds(row, 1)], rows_v.at[pl.ds(j, 1)], sem
                )
            )
        for cp in copies:
            cp.wait()
        pltpu.sync_copy(rows_v, out_hbm.at[pl.ds(base, _ROWS_PER_W)])


@jax.jit
def kernel(x, emb_mat):
    idx = x.reshape(-1).astype(jnp.int32)
    out = _sc_gather(emb_mat, idx)
    return out.reshape(1, SEQ, EMB)


# speedup vs baseline: 24.0400x; 24.0400x over previous
import functools

import jax
import jax.numpy as jnp
from jax import lax
from jax.experimental import pallas as pl
from jax.experimental.pallas import tpu as pltpu
from jax.experimental.pallas import tpu_sc as plsc

VOCAB_DIM = 1000000
EMB = 64
SEQ = 200

_info = plsc.get_sparse_core_info()
_NC, _NS = _info.num_cores, _info.num_subcores
_ROWS_PER_W = 8
_N_ACTIVE = SEQ // _ROWS_PER_W
_LANES = 128


@functools.partial(
    pl.kernel,
    mesh=plsc.VectorSubcoreMesh(core_axis_name="c", subcore_axis_name="s"),
    out_type=jax.ShapeDtypeStruct((SEQ, EMB), jnp.float32),
    scratch_types=[
        pltpu.VMEM((16,), jnp.int32),
        pltpu.VMEM((_ROWS_PER_W, EMB, _LANES), jnp.float32),
        pltpu.VMEM((_ROWS_PER_W, EMB), jnp.float32),
        pltpu.SemaphoreType.DMA,
    ],
    compiler_params=pltpu.CompilerParams(
        use_tc_tiling_on_sc=True, needs_layout_passes=False
    ),
)
def _sc_gather(table_hbm, idx_hbm, out_hbm, idx_v, tiles_v, rows_v, sem):
    wid = lax.axis_index("s") * _NC + lax.axis_index("c")

    @pl.when(wid < _N_ACTIVE)
    def _():
        base = wid * _ROWS_PER_W
        pltpu.sync_copy(
            idx_hbm.at[pl.ds(base, _ROWS_PER_W)], idx_v.at[pl.ds(0, _ROWS_PER_W)]
        )
        idx_vec = idx_v[...]
        copies = []
        for j in range(_ROWS_PER_W):
            tile_base = pl.multiple_of((idx_vec[j] // _LANES) * _LANES, _LANES)
            copies.append(
                pltpu.async_copy(
                    table_hbm.at[:, pl.ds(tile_base, _LANES)],
                    tiles_v.at[j],
                    sem,
                )
            )
        for cp in copies:
            cp.wait()
        lane_ids = jax.lax.iota(jnp.int32, 16)
        for j in range(_ROWS_PER_W):
            lane = idx_vec[j] % _LANES
            lane_vec = jnp.full((16,), lane, jnp.int32)
            for c in range(EMB // 16):
                g = plsc.load_gather(
                    tiles_v.at[j], [lane_ids + c * 16, lane_vec]
                )
                rows_v[j, pl.ds(c * 16, 16)] = g
        pltpu.sync_copy(rows_v, out_hbm.at[pl.ds(base, _ROWS_PER_W)])


@jax.jit
def kernel(x, emb_mat):
    idx = x.reshape(-1).astype(jnp.int32)
    out = _sc_gather(emb_mat.T, idx)
    return out.reshape(1, SEQ, EMB)


# speedup vs baseline: 24.2467x; 1.0086x over previous
import functools

import jax
import jax.numpy as jnp
from jax import lax
from jax.experimental import pallas as pl
from jax.experimental.pallas import tpu as pltpu
from jax.experimental.pallas import tpu_sc as plsc

VOCAB_DIM = 1000000
EMB = 64
SEQ = 200

_info = plsc.get_sparse_core_info()
_NC, _NS = _info.num_cores, _info.num_subcores
_ROWS_PER_W = 8
_N_ACTIVE = SEQ // _ROWS_PER_W
_LANES = 128


@functools.partial(
    pl.kernel,
    mesh=plsc.VectorSubcoreMesh(core_axis_name="c", subcore_axis_name="s"),
    out_type=jax.ShapeDtypeStruct((SEQ, EMB), jnp.float32),
    scratch_types=[
        pltpu.VMEM((16,), jnp.int32),
        pltpu.VMEM((_ROWS_PER_W, EMB, _LANES), jnp.float32),
        pltpu.VMEM((_ROWS_PER_W, EMB), jnp.float32),
        [pltpu.SemaphoreType.DMA] * _ROWS_PER_W,
    ],
    compiler_params=pltpu.CompilerParams(
        use_tc_tiling_on_sc=True, needs_layout_passes=False
    ),
)
def _sc_gather(table_hbm, idx_hbm, out_hbm, idx_v, tiles_v, rows_v, sems):
    wid = lax.axis_index("s") * _NC + lax.axis_index("c")

    @pl.when(wid < _N_ACTIVE)
    def _():
        base = wid * _ROWS_PER_W
        pltpu.sync_copy(
            idx_hbm.at[pl.ds(base, _ROWS_PER_W)], idx_v.at[pl.ds(0, _ROWS_PER_W)]
        )
        idx_vec = idx_v[...]
        copies = []
        for j in range(_ROWS_PER_W):
            tile_base = pl.multiple_of((idx_vec[j] // _LANES) * _LANES, _LANES)
            copies.append(
                pltpu.async_copy(
                    table_hbm.at[:, pl.ds(tile_base, _LANES)],
                    tiles_v.at[j],
                    sems[j],
                )
            )
        lane_ids = jax.lax.iota(jnp.int32, 16)
        for j in range(_ROWS_PER_W):
            copies[j].wait()
            lane = idx_vec[j] % _LANES
            lane_vec = jnp.full((16,), lane, jnp.int32)
            for c in range(EMB // 16):
                g = plsc.load_gather(
                    tiles_v.at[j], [lane_ids + c * 16, lane_vec]
                )
                rows_v[j, pl.ds(c * 16, 16)] = g
        pltpu.sync_copy(rows_v, out_hbm.at[pl.ds(base, _ROWS_PER_W)])


@jax.jit
def kernel(x, emb_mat):
    idx = x.reshape(-1).astype(jnp.int32)
    out = _sc_gather(emb_mat.T, idx)
    return out.reshape(1, SEQ, EMB)
